# baseline (speedup 1.0000x reference)
import numpy as np
import jax
from jax import lax
import jax.numpy as jnp
from jax.experimental import pallas as pl
from jax.experimental.pallas import tpu as pltpu
from jax.experimental.pallas import tpu_sc as plsc

_NB = 100
_N = 4096
_RMAX = float(5.0 * np.sqrt(1.0 / (np.pi * _N)))
_BW = _RMAX / _NB
_SIG = 0.25 * _RMAX
_LANES = 128

_BLK = 128
_NBLK = _N // _BLK
_RB = 1024
_NRB = _N // _RB
_NC = 2
_NS = 16
_ROWS_PER_TILE = _N // (_NC * _NS)
_HWORDS = _ROWS_PER_TILE * _LANES
_QSCALE = 65535.0
_RSKIP2 = float((_RMAX * (1.0 + 1e-4)) ** 2)


def _mask_body(xg_ref, yg_ref, xgt_ref, ygt_ref, m_ref):
    xg = xg_ref[...]
    yg = yg_ref[...]
    xgt = xgt_ref[...]
    ygt = ygt_ref[...]
    minxc = jnp.min(xg, axis=1, keepdims=True)
    maxxc = jnp.max(xg, axis=1, keepdims=True)
    minyc = jnp.min(yg, axis=1, keepdims=True)
    maxyc = jnp.max(yg, axis=1, keepdims=True)
    minxr = jnp.min(xgt, axis=0, keepdims=True)
    maxxr = jnp.max(xgt, axis=0, keepdims=True)
    minyr = jnp.min(ygt, axis=0, keepdims=True)
    maxyr = jnp.max(ygt, axis=0, keepdims=True)
    zero = jnp.zeros((_NBLK, _NBLK), jnp.float32)
    dxg = jnp.maximum(zero, jnp.maximum(minxc - maxxr, minxr - maxxc))
    dyg = jnp.maximum(zero, jnp.maximum(minyc - maxyr, minyr - maxyc))
    gap2 = dxg * dxg + dyg * dyg
    m_ref[...] = (gap2 < _RSKIP2).astype(jnp.int32)


def _pairs_body(cm_ref, fm_ref, xr_ref, yr_ref, xc_ref, yc_ref, p_hbm, b0, b1, s0, s1):
    i = pl.program_id(0)
    t = pl.program_id(1)
    linear = i * _NBLK + t
    last = _NRB * _NBLK - 1
    cond = cm_ref[i, t] != 0

    def cond_at(lin):
        linc = jnp.maximum(lin, 0)
        return (lin >= 0) & (cm_ref[linc // _NBLK, linc % _NBLK] != 0)

    def compute(buf, sem):
        for k in range(_RB // _BLK):

            @pl.when(fm_ref[i * (_RB // _BLK) + k, t] != 0)
            def _():
                xr = xr_ref[pl.ds(k * _BLK, _BLK), :]
                yr = yr_ref[pl.ds(k * _BLK, _BLK), :]
                xc = xc_ref[...]
                yc = yc_ref[...]
                dx = xr - xc
                dy = yr - yc
                d = jnp.sqrt(dx * dx + dy * dy + 1e-12)
                bidx = jnp.clip(jnp.floor(d / _BW), 0.0, float(_NB - 1))
                rc = (bidx + 0.5) * _BW
                ker = jnp.exp((d - rc) * (d - rc) * (-1.0 / (_SIG * _SIG)))
                row = (
                    jax.lax.broadcasted_iota(jnp.int32, (_BLK, _BLK), 0)
                    + (i * _RB + k * _BLK)
                )
                col = jax.lax.broadcasted_iota(jnp.int32, (_BLK, _BLK), 1) + t * _BLK
                valid = (d < _RMAX) & (row != col)
                w = jnp.where(valid, ker, 0.0)
                wq = jnp.floor(w * _QSCALE + 0.5).astype(jnp.int32)
                lrow = jax.lax.broadcasted_iota(jnp.int32, (_BLK, _BLK), 1)
                bin_eff = jnp.where(valid, bidx.astype(jnp.int32), 100 + (lrow & 15))
                f = (lrow * _LANES + bin_eff) * 65536 + wq
                buf[pl.ds(k * _BLK, _BLK), :] = f

        pltpu.make_async_copy(
            buf, p_hbm.at[t, pl.ds(i * _RB, _RB)], sem
        ).start()

    bufs = (b0, b1)
    sems = (s0, s1)
    for par in (0, 1):
        @pl.when((linear % 2 == par) & cond_at(linear - 2))
        def _():
            pltpu.make_async_copy(
                bufs[par], p_hbm.at[t, pl.ds(i * _RB, _RB)], sems[par]
            ).wait()

        @pl.when((linear % 2 == par) & cond)
        def _():
            compute(bufs[par], sems[par])

    for back in (1, 0):

        @pl.when((linear == last) & cond_at(linear - back))
        def _():
            pltpu.make_async_copy(
                bufs[(last - back) % 2],
                p_hbm.at[t, pl.ds(i * _RB, _RB)],
                sems[(last - back) % 2],
            ).wait()


def _sc_body(p_hbm, m_hbm, out_hbm, pb, hist, msk, red, sem):
    cid = lax.axis_index("c")
    sid = lax.axis_index("s")
    wid = sid * _NC + cid

    pltpu.sync_copy(m_hbm.at[wid], msk)

    def zero_body(i, _):
        hist[pl.ds(i * 16, 16)] = jnp.zeros((16,), jnp.float32)
        return 0

    lax.fori_loop(0, _HWORDS // 16, zero_body, 0)

    def process():
        @plsc.parallel_loop(0, _BLK, 1, unroll=4)
        def col_body(c):
            for g in range(_ROWS_PER_TILE // 16):
                v = pb[c, pl.ds(16 * g, 16)]
                idx = lax.shift_right_logical(v, 16)
                wq = v & 0xFFFF
                w = wq.astype(jnp.float32) * (1.0 / _QSCALE)
                plsc.addupdate_scatter(hist, [idx], w)

    for ci in range(_NBLK):
        mvec = msk[pl.ds((ci // 16) * 16, 16)]

        @pl.when(mvec[ci % 16] != 0)
        def _():
            pltpu.async_copy(
                p_hbm.at[wid, pl.ds(ci * _BLK, _BLK)], pb, sem
            ).wait()
            process()

    ninf = jnp.full((16,), -jnp.inf, jnp.float32)
    pinf = jnp.full((16,), jnp.inf, jnp.float32)
    zv = jnp.zeros((16,), jnp.float32)
    init = tuple([zv] * 8 + [pinf] * 8 + [ninf] * 8)

    def red_body(r, acc):
        acc = list(acc)
        for j in range(8):
            v = hist[pl.ds(r * _LANES + j * 16, 16)]
            acc[j] = acc[j] + v
            acc[8 + j] = jnp.minimum(acc[8 + j], v)
            acc[16 + j] = jnp.maximum(acc[16 + j], v)
        return tuple(acc)

    acc = lax.fori_loop(0, _ROWS_PER_TILE, red_body, init)
    for j in range(8):
        red[pl.ds(j * 16, 16)] = acc[j]
        red[pl.ds(_LANES + j * 16, 16)] = acc[8 + j]
        red[pl.ds(2 * _LANES + j * 16, 16)] = acc[16 + j]

    pltpu.sync_copy(red, out_hbm.at[pl.ds(wid * 3 * _LANES, 3 * _LANES)])


def _reduce_body(s_ref, mn_ref, mx_ref, scale_ref, out_ref):
    s = s_ref[...]
    mn = mn_ref[...]
    mx = mx_ref[...]
    scale = scale_ref[...]
    mean = jnp.sum(s, axis=0, keepdims=True) * (1.0 / _N) * scale
    mnr = jnp.min(mn, axis=0, keepdims=True) * scale
    mxr = jnp.max(mx, axis=0, keepdims=True) * scale
    out_ref[...] = jnp.concatenate(
        [mean, mnr, mxr, jnp.zeros((5, _LANES), jnp.float32)], axis=0
    )


def kernel(points):
    coords = points[:, :2].astype(jnp.float32)
    cell = jnp.floor(coords / _RMAX).astype(jnp.int32)
    cid = cell[:, 1] * 64 + cell[:, 0]
    keys = cid * _N + jnp.arange(_N, dtype=jnp.int32)
    perm = jnp.sort(keys) & (_N - 1)
    cs = coords[perm]

    xs = cs[:, 0].reshape(1, _N)
    ys = cs[:, 1].reshape(1, _N)
    xcol = cs[:, 0].reshape(_N, 1)
    ycol = cs[:, 1].reshape(_N, 1)
    xg = cs[:, 0].reshape(_NBLK, _BLK)
    yg = cs[:, 1].reshape(_NBLK, _BLK)
    xgt = jnp.transpose(xg)
    ygt = jnp.transpose(yg)

    mask = pl.pallas_call(
        _mask_body,
        grid=(1,),
        in_specs=[
            pl.BlockSpec((_NBLK, _BLK), lambda i: (0, 0)),
            pl.BlockSpec((_NBLK, _BLK), lambda i: (0, 0)),
            pl.BlockSpec((_BLK, _NBLK), lambda i: (0, 0)),
            pl.BlockSpec((_BLK, _NBLK), lambda i: (0, 0)),
        ],
        out_specs=pl.BlockSpec((_NBLK, _NBLK), lambda i: (0, 0)),
        out_shape=jax.ShapeDtypeStruct((_NBLK, _NBLK), jnp.int32),
    )(xg, yg, xgt, ygt)

    cmask = jnp.max(mask.reshape(_NRB, _RB // _BLK, _NBLK), axis=1)

    packed = pl.pallas_call(
        _pairs_body,
        grid=(_NRB, _NBLK),
        in_specs=[
            pl.BlockSpec(memory_space=pltpu.SMEM),
            pl.BlockSpec(memory_space=pltpu.SMEM),
            pl.BlockSpec((_RB, 1), lambda i, t: (i, 0)),
            pl.BlockSpec((_RB, 1), lambda i, t: (i, 0)),
            pl.BlockSpec((1, _BLK), lambda i, t: (0, t)),
            pl.BlockSpec((1, _BLK), lambda i, t: (0, t)),
        ],
        out_specs=pl.BlockSpec(memory_space=pl.ANY),
        out_shape=jax.ShapeDtypeStruct((_NBLK, _N, _BLK), jnp.int32),
        scratch_shapes=[
            pltpu.VMEM((_RB, _BLK), jnp.int32),
            pltpu.VMEM((_RB, _BLK), jnp.int32),
            pltpu.SemaphoreType.DMA,
            pltpu.SemaphoreType.DMA,
        ],
    )(cmask, mask, xcol, ycol, xs, ys)

    mesh = plsc.VectorSubcoreMesh(
        core_axis_name="c", subcore_axis_name="s", num_cores=_NC, num_subcores=_NS
    )
    partials = pl.kernel(
        _sc_body,
        out_type=jax.ShapeDtypeStruct((_NBLK * 3 * _LANES,), jnp.float32),
        mesh=mesh,
        compiler_params=pltpu.CompilerParams(
            use_tc_tiling_on_sc=False, needs_layout_passes=False
        ),
        scratch_types=[
            pltpu.VMEM((_BLK, _BLK), jnp.int32),
            pltpu.VMEM((_HWORDS,), jnp.float32),
            pltpu.VMEM((_NBLK,), jnp.int32),
            pltpu.VMEM((3 * _LANES,), jnp.float32),
            pltpu.SemaphoreType.DMA,
        ],
    )(packed, mask)

    p3 = partials.reshape(_NBLK, 3, _LANES)
    sums = p3[:, 0, :]
    mins = p3[:, 1, :]
    maxs = p3[:, 2, :]

    k = np.arange(_LANES, dtype=np.float64)
    ring_area = np.pi * (((k + 1.0) * _BW) ** 2 - (k * _BW) ** 2)
    scale = (1.0 / (ring_area * float(_N))).astype(np.float32).reshape(1, _LANES)

    red = pl.pallas_call(
        _reduce_body,
        grid=(1,),
        in_specs=[
            pl.BlockSpec((_NBLK, _LANES), lambda i: (0, 0)),
            pl.BlockSpec((_NBLK, _LANES), lambda i: (0, 0)),
            pl.BlockSpec((_NBLK, _LANES), lambda i: (0, 0)),
            pl.BlockSpec((1, _LANES), lambda i: (0, 0)),
        ],
        out_specs=pl.BlockSpec((8, _LANES), lambda i: (0, 0)),
        out_shape=jax.ShapeDtypeStruct((8, _LANES), jnp.float32),
    )(sums, mins, maxs, jnp.asarray(scale))

    kk = np.arange(_NB, dtype=np.float64)
    rs = jnp.asarray(((kk + 0.5) * _BW / _RMAX).astype(np.float32))
    return jnp.stack([rs, red[0, :_NB], red[1, :_NB], red[2, :_NB]], axis=1)


# speedup vs baseline: 1.0665x; 1.0665x over previous
import numpy as np
import jax
from jax import lax
import jax.numpy as jnp
from jax.experimental import pallas as pl
from jax.experimental.pallas import tpu as pltpu
from jax.experimental.pallas import tpu_sc as plsc

_NB = 100
_N = 4096
_RMAX = float(5.0 * np.sqrt(1.0 / (np.pi * _N)))
_BW = _RMAX / _NB
_SIG = 0.25 * _RMAX
_LANES = 128

_BLK = 128
_NBLK = _N // _BLK
_RB = 1024
_NRB = _N // _RB
_NC = 2
_NS = 16
_ROWS_PER_TILE = _N // (_NC * _NS)
_HWORDS = _ROWS_PER_TILE * _LANES
_QSCALE = 65535.0
_RSKIP2 = float((_RMAX * (1.0 + 1e-4)) ** 2)


def _mask_body(xg_ref, yg_ref, xgt_ref, ygt_ref, m_ref):
    xg = xg_ref[...]
    yg = yg_ref[...]
    xgt = xgt_ref[...]
    ygt = ygt_ref[...]
    minxc = jnp.min(xg, axis=1, keepdims=True)
    maxxc = jnp.max(xg, axis=1, keepdims=True)
    minyc = jnp.min(yg, axis=1, keepdims=True)
    maxyc = jnp.max(yg, axis=1, keepdims=True)
    minxr = jnp.min(xgt, axis=0, keepdims=True)
    maxxr = jnp.max(xgt, axis=0, keepdims=True)
    minyr = jnp.min(ygt, axis=0, keepdims=True)
    maxyr = jnp.max(ygt, axis=0, keepdims=True)
    zero = jnp.zeros((_NBLK, _NBLK), jnp.float32)
    dxg = jnp.maximum(zero, jnp.maximum(minxc - maxxr, minxr - maxxc))
    dyg = jnp.maximum(zero, jnp.maximum(minyc - maxyr, minyr - maxyc))
    gap2 = dxg * dxg + dyg * dyg
    m_ref[...] = (gap2 < _RSKIP2).astype(jnp.int32)


def _pairs_body(cm_ref, fm_ref, xr_ref, yr_ref, xc_ref, yc_ref, p_hbm, b0, b1, s0, s1):
    i = pl.program_id(0)
    t = pl.program_id(1)
    linear = i * _NBLK + t
    last = _NRB * _NBLK - 1
    cond = cm_ref[i, t] != 0

    def cond_at(lin):
        linc = jnp.maximum(lin, 0)
        return (lin >= 0) & (cm_ref[linc // _NBLK, linc % _NBLK] != 0)

    def compute(buf, sem):
        for k in range(_RB // _BLK):

            @pl.when(fm_ref[i * (_RB // _BLK) + k, t] != 0)
            def _():
                xr = xr_ref[pl.ds(k * _BLK, _BLK), :]
                yr = yr_ref[pl.ds(k * _BLK, _BLK), :]
                xc = xc_ref[...]
                yc = yc_ref[...]
                dx = xr - xc
                dy = yr - yc
                d = jnp.sqrt(dx * dx + dy * dy + 1e-12)
                bidx = jnp.clip(jnp.floor(d / _BW), 0.0, float(_NB - 1))
                rc = (bidx + 0.5) * _BW
                ker = jnp.exp((d - rc) * (d - rc) * (-1.0 / (_SIG * _SIG)))
                row = (
                    jax.lax.broadcasted_iota(jnp.int32, (_BLK, _BLK), 0)
                    + (i * _RB + k * _BLK)
                )
                col = jax.lax.broadcasted_iota(jnp.int32, (_BLK, _BLK), 1) + t * _BLK
                valid = (d < _RMAX) & (row != col)
                w = jnp.where(valid, ker, 0.0)
                wq = jnp.floor(w * _QSCALE + 0.5).astype(jnp.int32)
                lrow = jax.lax.broadcasted_iota(jnp.int32, (_BLK, _BLK), 1)
                bin_eff = jnp.where(valid, bidx.astype(jnp.int32), 100 + (lrow & 15))
                f = (lrow * _LANES + bin_eff) * 65536 + wq
                buf[pl.ds(k * _BLK, _BLK), :] = f

        pltpu.make_async_copy(
            buf, p_hbm.at[t, pl.ds(i * _RB, _RB)], sem
        ).start()

    bufs = (b0, b1)
    sems = (s0, s1)
    for par in (0, 1):
        @pl.when((linear % 2 == par) & cond_at(linear - 2))
        def _():
            pltpu.make_async_copy(
                bufs[par], p_hbm.at[t, pl.ds(i * _RB, _RB)], sems[par]
            ).wait()

        @pl.when((linear % 2 == par) & cond)
        def _():
            compute(bufs[par], sems[par])

    for back in (1, 0):

        @pl.when((linear == last) & cond_at(linear - back))
        def _():
            pltpu.make_async_copy(
                bufs[(last - back) % 2],
                p_hbm.at[t, pl.ds(i * _RB, _RB)],
                sems[(last - back) % 2],
            ).wait()


def _sc_body(p_hbm, m_hbm, out_hbm, pb0, pb1, hist, msk, red, s0, s1):
    cid = lax.axis_index("c")
    sid = lax.axis_index("s")
    wid = sid * _NC + cid

    pltpu.sync_copy(m_hbm.at[wid], msk)

    def zero_body(i, _):
        hist[pl.ds(i * 64, 16)] = jnp.zeros((16,), jnp.float32)
        hist[pl.ds(i * 64 + 16, 16)] = jnp.zeros((16,), jnp.float32)
        hist[pl.ds(i * 64 + 32, 16)] = jnp.zeros((16,), jnp.float32)
        hist[pl.ds(i * 64 + 48, 16)] = jnp.zeros((16,), jnp.float32)
        return 0

    lax.fori_loop(0, _HWORDS // 64, zero_body, 0)

    bufs = (pb0, pb1)
    sems = (s0, s1)

    def mask_at(ci):
        return msk[pl.ds((ci // 16) * 16, 16)][ci % 16] != 0

    def start(ci):
        @pl.when(mask_at(ci))
        def _():
            pltpu.async_copy(
                p_hbm.at[wid, pl.ds(ci * _BLK, _BLK)], bufs[ci % 2], sems[ci % 2]
            )

    def process(pb):
        @plsc.parallel_loop(0, _BLK, 1, unroll=2)
        def col_body(c):
            for g in range(_ROWS_PER_TILE // 16):
                v = pb[c, pl.ds(16 * g, 16)]
                idx = lax.shift_right_logical(v, 16)
                wq = v & 0xFFFF
                w = wq.astype(jnp.float32) * (1.0 / _QSCALE)
                plsc.addupdate_scatter(hist, [idx], w)

    start(0)
    for ci in range(_NBLK):
        if ci + 1 < _NBLK:
            start(ci + 1)

        @pl.when(mask_at(ci))
        def _():
            pltpu.make_async_copy(
                p_hbm.at[wid, pl.ds(ci * _BLK, _BLK)], bufs[ci % 2], sems[ci % 2]
            ).wait()
            process(bufs[ci % 2])

    ninf = jnp.full((16,), -jnp.inf, jnp.float32)
    pinf = jnp.full((16,), jnp.inf, jnp.float32)
    zv = jnp.zeros((16,), jnp.float32)
    init = tuple([zv] * 8 + [pinf] * 8 + [ninf] * 8)

    def red_body(r, acc):
        acc = list(acc)
        for j in range(8):
            v = hist[pl.ds(r * _LANES + j * 16, 16)]
            acc[j] = acc[j] + v
            acc[8 + j] = jnp.minimum(acc[8 + j], v)
            acc[16 + j] = jnp.maximum(acc[16 + j], v)
        return tuple(acc)

    acc = lax.fori_loop(0, _ROWS_PER_TILE, red_body, init)
    for j in range(8):
        red[pl.ds(j * 16, 16)] = acc[j]
        red[pl.ds(_LANES + j * 16, 16)] = acc[8 + j]
        red[pl.ds(2 * _LANES + j * 16, 16)] = acc[16 + j]

    pltpu.sync_copy(red, out_hbm.at[pl.ds(wid * 3 * _LANES, 3 * _LANES)])


def _reduce_body(s_ref, mn_ref, mx_ref, scale_ref, out_ref):
    s = s_ref[...]
    mn = mn_ref[...]
    mx = mx_ref[...]
    scale = scale_ref[...]
    mean = jnp.sum(s, axis=0, keepdims=True) * (1.0 / _N) * scale
    mnr = jnp.min(mn, axis=0, keepdims=True) * scale
    mxr = jnp.max(mx, axis=0, keepdims=True) * scale
    out_ref[...] = jnp.concatenate(
        [mean, mnr, mxr, jnp.zeros((5, _LANES), jnp.float32)], axis=0
    )


def kernel(points):
    coords = points[:, :2].astype(jnp.float32)
    cell = jnp.floor(coords / _RMAX).astype(jnp.int32)
    cid = cell[:, 1] * 64 + cell[:, 0]
    keys = cid * _N + jnp.arange(_N, dtype=jnp.int32)
    perm = jnp.sort(keys) & (_N - 1)
    cs = coords[perm]

    xs = cs[:, 0].reshape(1, _N)
    ys = cs[:, 1].reshape(1, _N)
    xcol = cs[:, 0].reshape(_N, 1)
    ycol = cs[:, 1].reshape(_N, 1)
    xg = cs[:, 0].reshape(_NBLK, _BLK)
    yg = cs[:, 1].reshape(_NBLK, _BLK)
    xgt = jnp.transpose(xg)
    ygt = jnp.transpose(yg)

    mask = pl.pallas_call(
        _mask_body,
        grid=(1,),
        in_specs=[
            pl.BlockSpec((_NBLK, _BLK), lambda i: (0, 0)),
            pl.BlockSpec((_NBLK, _BLK), lambda i: (0, 0)),
            pl.BlockSpec((_BLK, _NBLK), lambda i: (0, 0)),
            pl.BlockSpec((_BLK, _NBLK), lambda i: (0, 0)),
        ],
        out_specs=pl.BlockSpec((_NBLK, _NBLK), lambda i: (0, 0)),
        out_shape=jax.ShapeDtypeStruct((_NBLK, _NBLK), jnp.int32),
    )(xg, yg, xgt, ygt)

    cmask = jnp.max(mask.reshape(_NRB, _RB // _BLK, _NBLK), axis=1)

    packed = pl.pallas_call(
        _pairs_body,
        grid=(_NRB, _NBLK),
        in_specs=[
            pl.BlockSpec(memory_space=pltpu.SMEM),
            pl.BlockSpec(memory_space=pltpu.SMEM),
            pl.BlockSpec((_RB, 1), lambda i, t: (i, 0)),
            pl.BlockSpec((_RB, 1), lambda i, t: (i, 0)),
            pl.BlockSpec((1, _BLK), lambda i, t: (0, t)),
            pl.BlockSpec((1, _BLK), lambda i, t: (0, t)),
        ],
        out_specs=pl.BlockSpec(memory_space=pl.ANY),
        out_shape=jax.ShapeDtypeStruct((_NBLK, _N, _BLK), jnp.int32),
        scratch_shapes=[
            pltpu.VMEM((_RB, _BLK), jnp.int32),
            pltpu.VMEM((_RB, _BLK), jnp.int32),
            pltpu.SemaphoreType.DMA,
            pltpu.SemaphoreType.DMA,
        ],
    )(cmask, mask, xcol, ycol, xs, ys)

    mesh = plsc.VectorSubcoreMesh(
        core_axis_name="c", subcore_axis_name="s", num_cores=_NC, num_subcores=_NS
    )
    partials = pl.kernel(
        _sc_body,
        out_type=jax.ShapeDtypeStruct((_NBLK * 3 * _LANES,), jnp.float32),
        mesh=mesh,
        compiler_params=pltpu.CompilerParams(
            use_tc_tiling_on_sc=False, needs_layout_passes=False
        ),
        scratch_types=[
            pltpu.VMEM((_BLK, _BLK), jnp.int32),
            pltpu.VMEM((_BLK, _BLK), jnp.int32),
            pltpu.VMEM((_HWORDS,), jnp.float32),
            pltpu.VMEM((_NBLK,), jnp.int32),
            pltpu.VMEM((3 * _LANES,), jnp.float32),
            pltpu.SemaphoreType.DMA,
            pltpu.SemaphoreType.DMA,
        ],
    )(packed, mask)

    p3 = partials.reshape(_NBLK, 3, _LANES)
    sums = p3[:, 0, :]
    mins = p3[:, 1, :]
    maxs = p3[:, 2, :]

    k = np.arange(_LANES, dtype=np.float64)
    ring_area = np.pi * (((k + 1.0) * _BW) ** 2 - (k * _BW) ** 2)
    scale = (1.0 / (ring_area * float(_N))).astype(np.float32).reshape(1, _LANES)

    red = pl.pallas_call(
        _reduce_body,
        grid=(1,),
        in_specs=[
            pl.BlockSpec((_NBLK, _LANES), lambda i: (0, 0)),
            pl.BlockSpec((_NBLK, _LANES), lambda i: (0, 0)),
            pl.BlockSpec((_NBLK, _LANES), lambda i: (0, 0)),
            pl.BlockSpec((1, _LANES), lambda i: (0, 0)),
        ],
        out_specs=pl.BlockSpec((8, _LANES), lambda i: (0, 0)),
        out_shape=jax.ShapeDtypeStruct((8, _LANES), jnp.float32),
    )(sums, mins, maxs, jnp.asarray(scale))

    kk = np.arange(_NB, dtype=np.float64)
    rs = jnp.asarray(((kk + 0.5) * _BW / _RMAX).astype(np.float32))
    return jnp.stack([rs, red[0, :_NB], red[1, :_NB], red[2, :_NB]], axis=1)
